# Initial kernel scaffold; baseline (speedup 1.0000x reference)
import jax
import jax.numpy as jnp
from jax.experimental import pallas as pl


def kernel(x, pos, edge_index, W1, Wr1, W2, Wr2, Wh1, bh1, Wh2, bh2):
    raise NotImplementedError("write your pallas kernel here")


# speedup vs baseline: 3.7998x; 3.7998x over previous
import functools

import jax
import jax.numpy as jnp
from jax import lax
from jax.experimental import pallas as pl
from jax.experimental.pallas import tpu as pltpu
from jax.experimental.pallas import tpu_sc as plsc

N = 50000
E = 800000
D = 64
NB = 8
R_MAX = 5.0

NC = 2
NS = 16
L = 16

f32 = jnp.float32
bf16 = jnp.bfloat16
i32 = jnp.int32


EPG = E // (NC * NS)
CG = 1000
GSUB = 40
NCHUNK_G = EPG // CG

@functools.cache
def _make_sc_geom():
    mesh = plsc.VectorSubcoreMesh(core_axis_name="c", subcore_axis_name="s")
    return functools.partial(
        pl.kernel,
        mesh=mesh,
        out_type=jax.ShapeDtypeStruct((E,), f32),
        scratch_types=[
            pltpu.VMEM((CG,), i32),
            pltpu.VMEM((CG,), i32),
            pltpu.VMEM((CG, 16), f32),
            pltpu.VMEM((CG, 16), f32),
            pltpu.VMEM((CG,), f32),
            pltpu.SemaphoreType.DMA,
        ],
        compiler_params=pltpu.CompilerParams(
            needs_layout_passes=False, use_tc_tiling_on_sc=False),
    )(_sc_geom_body)


def _sc_geom_body(pos16_hbm, src_hbm, dst_hbm, d2_hbm, srcv, dstv, psrc, pdst,
                  d2buf, sem):
    c = lax.axis_index("c")
    s = lax.axis_index("s")
    wid = s * NC + c
    base = wid * EPG

    @pl.loop(0, NCHUNK_G)
    def _chunk(i):
        eoff = base + i * CG
        cps = []
        for j in range(0, CG, GSUB):
            pltpu.sync_copy(src_hbm.at[pl.ds(eoff + j, GSUB)],
                            srcv.at[pl.ds(j, GSUB)])
            pltpu.sync_copy(dst_hbm.at[pl.ds(eoff + j, GSUB)],
                            dstv.at[pl.ds(j, GSUB)])
            cps.append(pltpu.async_copy(
                pos16_hbm.at[srcv.at[pl.ds(j, GSUB)]],
                psrc.at[pl.ds(j, GSUB)], sem))
            cps.append(pltpu.async_copy(
                pos16_hbm.at[dstv.at[pl.ds(j, GSUB)]],
                pdst.at[pl.ds(j, GSUB)], sem))
        for cp in cps:
            cp.wait()

        @pl.loop(0, CG, step=16)
        def _grp(m):
            rows = m + lax.iota(i32, 16)
            col0 = jnp.zeros((16,), i32)
            col1 = col0 + 1
            col2 = col0 + 2
            dx = (plsc.load_gather(pdst, [rows, col0])
                  - plsc.load_gather(psrc, [rows, col0]))
            dy = (plsc.load_gather(pdst, [rows, col1])
                  - plsc.load_gather(psrc, [rows, col1]))
            dz = (plsc.load_gather(pdst, [rows, col2])
                  - plsc.load_gather(psrc, [rows, col2]))
            d2buf[pl.ds(m, 16)] = dx * dx + dy * dy + dz * dz

        pltpu.sync_copy(d2buf, d2_hbm.at[pl.ds(eoff, CG)])


HALF = N // 2
RPS = 1568
HALF_PAD = NS * RPS
EPW = E // NS
CA = 80
NCHUNK_A = EPW // CA
WPS = 1562

@functools.cache
def _make_sc_agg():
    mesh = plsc.VectorSubcoreMesh(core_axis_name="c", subcore_axis_name="s")
    return functools.partial(
        pl.kernel,
        mesh=mesh,
        out_type=jax.ShapeDtypeStruct((N, D), bf16),
        scratch_types=[
            pltpu.VMEM_SHARED((HALF_PAD, D), bf16),
            pltpu.VMEM((2, CA), i32),
            pltpu.VMEM((2, CA), i32),
            pltpu.VMEM((CA,), i32),
            pltpu.VMEM((CA,), i32),
            pltpu.VMEM((CA, D), bf16),
            pltpu.VMEM((CA, D), bf16),
            pltpu.VMEM((CA, D), bf16),
            pltpu.VMEM((CA, D), bf16),
            pltpu.SemaphoreType.DMA,
            pltpu.SemaphoreType.DMA,
        ],
        compiler_params=pltpu.CompilerParams(
            needs_layout_passes=False, use_tc_tiling_on_sc=False),
    )(_sc_agg_body)


def _sc_agg_body(hx_hbm, coef_hbm, ei_hbm, out_hbm,
                 acc, ij0, ij1, dl0, dl1, hs0, hs1, cf0, cf1, sem0, sem1):
    c = lax.axis_index("c")
    s = lax.axis_index("s")
    base_node = c * HALF
    ijs, dls, hss, cfs, sems = ((ij0, ij1), (dl0, dl1), (hs0, hs1),
                                (cf0, cf1), (sem0, sem1))

    @pl.loop(0, CA)
    def _z(r):
        for k in range(0, D, 32):
            hs0[r, pl.ds(k, 32)] = jnp.zeros((32,), bf16)

    row0 = s * RPS
    for off in range(0, RPS, CA):
        sz = min(CA, RPS - off)
        pltpu.sync_copy(hs0.at[pl.ds(0, sz)], acc.at[pl.ds(row0 + off, sz)])
    plsc.subcore_barrier()

    def start(ic, b):
        eoff = s * EPW + ic * CA
        pltpu.sync_copy(ei_hbm.at[:, pl.ds(eoff, CA)], ijs[b])
        pltpu.async_copy(hx_hbm.at[ijs[b].at[0]], hss[b], sems[b])
        pltpu.async_copy(coef_hbm.at[pl.ds(eoff, CA)], cfs[b], sems[b])

    def finish(ic, b):
        ij, dl, hs, cf = ijs[b], dls[b], hss[b], cfs[b]
        pltpu.make_async_copy(hx_hbm.at[ij.at[0]], hs, sems[b]).wait()
        pltpu.make_async_copy(coef_hbm.at[pl.ds(0, CA)], cf, sems[b]).wait()

        @pl.loop(0, CA, step=16)
        def _remap(k):
            dvec = ij[1, pl.ds(k, 16)]
            dli = dvec - base_node
            ok = (dli >= 0) & (dli < HALF)
            dl[pl.ds(k, 16)] = jnp.where(ok, dli, HALF)

        @pl.loop(0, CA)
        def _mul(r):
            for k in range(0, D, 32):
                hs[r, pl.ds(k, 32)] = hs[r, pl.ds(k, 32)] * cf[r, pl.ds(k, 32)]

        pltpu.sync_copy(hs, acc.at[dl], add=True)

    start(0, 0)

    @pl.loop(0, (NCHUNK_A - 1) // 2)
    def _pair(k):
        ic = 2 * k
        start(ic + 1, 1)
        finish(ic, 0)
        start(ic + 2, 0)
        finish(ic + 1, 1)

    finish(NCHUNK_A - 1, 0)
    plsc.subcore_barrier()

    w0 = s * WPS
    pltpu.sync_copy(acc.at[pl.ds(w0, WPS)],
                    out_hbm.at[pl.ds(base_node + w0, WPS)])

    @pl.when(s == 0)
    def _tail():
        pltpu.sync_copy(acc.at[pl.ds(NS * WPS, HALF - NS * WPS)],
                        out_hbm.at[pl.ds(base_node + NS * WPS,
                                         HALF - NS * WPS)])


BN = 2000
BE = 32000
NBLK_E = E // BE

_GAMMA = (NB / R_MAX) ** 2
_CENTERS = tuple(float(v) for v in
                 (R_MAX * k / (NB - 1) for k in range(NB)))


def _tc_matmul_body(x_ref, w_ref, o_ref):
    o_ref[...] = jnp.dot(x_ref[...], w_ref[...],
                         preferred_element_type=f32).astype(bf16)


def _tc_matmul(x, w):
    return pl.pallas_call(
        _tc_matmul_body,
        grid=(N // BN,),
        in_specs=[pl.BlockSpec((BN, D), lambda i: (i, 0)),
                  pl.BlockSpec((D, D), lambda i: (0, 0))],
        out_specs=pl.BlockSpec((BN, D), lambda i: (i, 0)),
        out_shape=jax.ShapeDtypeStruct((N, D), bf16),
    )(x, w)


def _tc_coef_body(d2_ref, wr1_ref, wr2_ref, c1_ref, c2_ref):
    d2 = d2_ref[0]
    dist = jnp.sqrt(d2 + 1e-12)
    dc = jnp.minimum(dist, R_MAX)
    cut = jnp.where(dist < R_MAX,
                    0.5 * (jnp.cos(jnp.pi * dc / R_MAX) + 1.0),
                    0.0)
    cen = (lax.broadcasted_iota(i32, (NB, 1), 0).astype(f32)
           * (R_MAX / (NB - 1)))
    rb = jnp.exp(-_GAMMA * (dist - cen) ** 2)
    pt = rb * cut
    dn = (((0,), (0,)), ((), ()))
    c1_ref[...] = lax.dot_general(pt, wr1_ref[...], dn,
                                  preferred_element_type=f32).astype(bf16)
    c2_ref[...] = lax.dot_general(pt, wr2_ref[...], dn,
                                  preferred_element_type=f32).astype(bf16)


def _tc_coef(d2, wr1, wr2):
    return pl.pallas_call(
        _tc_coef_body,
        grid=(NBLK_E,),
        in_specs=[pl.BlockSpec((1, 1, BE), lambda i: (i, 0, 0)),
                  pl.BlockSpec((NB, D), lambda i: (0, 0)),
                  pl.BlockSpec((NB, D), lambda i: (0, 0))],
        out_specs=[pl.BlockSpec((BE, D), lambda i: (i, 0)),
                   pl.BlockSpec((BE, D), lambda i: (i, 0))],
        out_shape=[jax.ShapeDtypeStruct((E, D), bf16),
                   jax.ShapeDtypeStruct((E, D), bf16)],
    )(d2.reshape(NBLK_E, 1, BE), wr1, wr2)


def _silu(v):
    return v / (1.0 + jnp.exp(-v))


def _tc_hnext_body(x_ref, agg_ref, w_ref, h1_ref, hx2_ref):
    v = x_ref[...] + agg_ref[...].astype(f32)
    h1 = _silu(v)
    h1_ref[...] = h1
    hx2_ref[...] = jnp.dot(h1, w_ref[...],
                           preferred_element_type=f32).astype(bf16)


def _tc_hnext(x, agg, w2):
    return pl.pallas_call(
        _tc_hnext_body,
        grid=(N // BN,),
        in_specs=[pl.BlockSpec((BN, D), lambda i: (i, 0)),
                  pl.BlockSpec((BN, D), lambda i: (i, 0)),
                  pl.BlockSpec((D, D), lambda i: (0, 0))],
        out_specs=[pl.BlockSpec((BN, D), lambda i: (i, 0)),
                   pl.BlockSpec((BN, D), lambda i: (i, 0))],
        out_shape=[jax.ShapeDtypeStruct((N, D), f32),
                   jax.ShapeDtypeStruct((N, D), bf16)],
    )(x, agg, w2)


def _tc_head_body(h1_ref, agg_ref, wh1_ref, bh1_ref, wh2_ref, bh2_ref, o_ref):
    v = h1_ref[...] + agg_ref[...].astype(f32)
    h2 = _silu(v)
    t = jnp.dot(h2, wh1_ref[...], preferred_element_type=f32) + bh1_ref[...]
    t = _silu(t)
    o_ref[...] = (jnp.dot(t, wh2_ref[...], preferred_element_type=f32)
                  + bh2_ref[...])


def _tc_head(h1, agg2, wh1, bh1, wh2, bh2):
    return pl.pallas_call(
        _tc_head_body,
        grid=(N // BN,),
        in_specs=[pl.BlockSpec((BN, D), lambda i: (i, 0)),
                  pl.BlockSpec((BN, D), lambda i: (i, 0)),
                  pl.BlockSpec((D, D // 2), lambda i: (0, 0)),
                  pl.BlockSpec((1, D // 2), lambda i: (0, 0)),
                  pl.BlockSpec((D // 2, 3), lambda i: (0, 0)),
                  pl.BlockSpec((1, 3), lambda i: (0, 0))],
        out_specs=pl.BlockSpec((BN, 3), lambda i: (i, 0)),
        out_shape=jax.ShapeDtypeStruct((N, 3), f32),
    )(h1, agg2, wh1, bh1, wh2, bh2)


def kernel(x, pos, edge_index, W1, Wr1, W2, Wr2, Wh1, bh1, Wh2, bh2):
    ei = edge_index.astype(i32)
    src = ei[0]
    dst = ei[1]
    pos16 = jnp.pad(pos.astype(f32), ((0, 0), (0, 13)))

    sc_geom = _make_sc_geom()
    sc_agg = _make_sc_agg()

    d2 = sc_geom(pos16, src, dst)
    coef1, coef2 = _tc_coef(d2, Wr1, Wr2)

    hx1 = _tc_matmul(x, W1)
    agg1 = sc_agg(hx1, coef1, ei)
    h1, hx2 = _tc_hnext(x, agg1, W2)
    agg2 = sc_agg(hx2, coef2, ei)
    return _tc_head(h1, agg2, Wh1, jnp.reshape(bh1, (1, D // 2)),
                    Wh2, jnp.reshape(bh2, (1, 3)))
